# baseline (device time: 1507815 ns/iter reference)
import jax
import jax.numpy as jnp
from jax import lax
from jax.experimental import pallas as pl
from jax.experimental.pallas import tpu as pltpu

N_DEV = 32


def kernel(x, w_mat):
    m_per, k = x.shape
    _, n_per = w_mat.shape
    m_total = N_DEV * m_per

    def body(x_ref, w_ref, out_ref, comm_ref, w_bf_ref, send_sems, recv_sems):
        my = lax.axis_index("i")
        left = lax.rem(my - 1 + N_DEV, N_DEV)
        right = lax.rem(my + 1, N_DEV)

        barrier_sem = pltpu.get_barrier_semaphore()
        for nbr in (left, right):
            pl.semaphore_signal(
                barrier_sem, inc=1,
                device_id=(nbr,), device_id_type=pl.DeviceIdType.MESH,
            )
        pl.semaphore_wait(barrier_sem, 2)

        w_bf_ref[...] = w_ref[...].astype(jnp.bfloat16)
        comm_ref[0] = x_ref[...].astype(jnp.bfloat16)

        acc = jnp.dot(comm_ref[0], w_bf_ref[...],
                      preferred_element_type=jnp.float32)
        out_ref[pl.ds(my * m_per, m_per), :] = jnp.maximum(acc, 0.0)

        for h in range(N_DEV - 1):
            send_slot = h % 2
            recv_slot = (h + 1) % 2
            rdma = pltpu.make_async_remote_copy(
                src_ref=comm_ref.at[send_slot],
                dst_ref=comm_ref.at[recv_slot],
                send_sem=send_sems.at[send_slot],
                recv_sem=recv_sems.at[recv_slot],
                device_id=(right,),
                device_id_type=pl.DeviceIdType.MESH,
            )
            rdma.start()
            rdma.wait()

            origin = lax.rem(my - h - 1 + N_DEV, N_DEV)
            blk = jnp.dot(comm_ref[recv_slot], w_bf_ref[...],
                          preferred_element_type=jnp.float32)
            out_ref[pl.ds(origin * m_per, m_per), :] = jnp.maximum(blk, 0.0)

    return pl.pallas_call(
        body,
        out_shape=jax.ShapeDtypeStruct((m_total, n_per), jnp.float32),
        in_specs=[
            pl.BlockSpec(memory_space=pltpu.VMEM),
            pl.BlockSpec(memory_space=pltpu.VMEM),
        ],
        out_specs=pl.BlockSpec(memory_space=pltpu.VMEM),
        scratch_shapes=[
            pltpu.VMEM((2, m_per, k), jnp.bfloat16),
            pltpu.VMEM((k, n_per), jnp.bfloat16),
            pltpu.SemaphoreType.DMA((2,)),
            pltpu.SemaphoreType.DMA((2,)),
        ],
        compiler_params=pltpu.CompilerParams(collective_id=0),
    )(x, w_mat)


# device time: 1476546 ns/iter; 1.0212x vs baseline; 1.0212x over previous
import jax
import jax.numpy as jnp
from jax import lax
from jax.experimental import pallas as pl
from jax.experimental.pallas import tpu as pltpu

N_DEV = 32
R_STEPS = 16
L_STEPS = 15
SLOTS = 3


def kernel(x, w_mat):
    m_per, k = x.shape
    _, n_per = w_mat.shape
    m_total = N_DEV * m_per

    def body(x_ref, w_ref, out_ref, comm_r, comm_l, w_bf,
             send_r, recv_r, send_l, recv_l):
        my = lax.axis_index("i")
        left = lax.rem(my - 1 + N_DEV, N_DEV)
        right = lax.rem(my + 1, N_DEV)

        barrier_sem = pltpu.get_barrier_semaphore()
        for nbr in (left, right):
            pl.semaphore_signal(
                barrier_sem, inc=1,
                device_id=(nbr,), device_id_type=pl.DeviceIdType.MESH,
            )
        pl.semaphore_wait(barrier_sem, 2)

        w_bf[...] = w_ref[...].astype(jnp.bfloat16)
        xb = x_ref[...].astype(jnp.bfloat16)
        comm_r[0] = xb
        comm_l[0] = xb

        def rdma_r(s):
            return pltpu.make_async_remote_copy(
                src_ref=comm_r.at[s % SLOTS],
                dst_ref=comm_r.at[(s + 1) % SLOTS],
                send_sem=send_r.at[s % SLOTS],
                recv_sem=recv_r.at[(s + 1) % SLOTS],
                device_id=(right,),
                device_id_type=pl.DeviceIdType.MESH,
            )

        def rdma_l(s):
            return pltpu.make_async_remote_copy(
                src_ref=comm_l.at[s % SLOTS],
                dst_ref=comm_l.at[(s + 1) % SLOTS],
                send_sem=send_l.at[s % SLOTS],
                recv_sem=recv_l.at[(s + 1) % SLOTS],
                device_id=(left,),
                device_id_type=pl.DeviceIdType.MESH,
            )

        def compute_band(chunk, origin):
            blk = jnp.dot(chunk, w_bf[...], preferred_element_type=jnp.float32)
            out_ref[pl.ds(origin * m_per, m_per), :] = jnp.maximum(blk, 0.0)

        for s in range(R_STEPS):
            rr = rdma_r(s)
            rr.start()
            rl = None
            if s < L_STEPS:
                rl = rdma_l(s)
                rl.start()

            if s == 0:
                compute_band(comm_r[0], my)
            else:
                compute_band(comm_r[s % SLOTS], lax.rem(my - s + N_DEV, N_DEV))
                compute_band(comm_l[s % SLOTS], lax.rem(my + s, N_DEV))

            rr.wait()
            if rl is not None:
                rl.wait()

        compute_band(comm_r[R_STEPS % SLOTS], lax.rem(my - R_STEPS + N_DEV, N_DEV))

    return pl.pallas_call(
        body,
        out_shape=jax.ShapeDtypeStruct((m_total, n_per), jnp.float32),
        in_specs=[
            pl.BlockSpec(memory_space=pltpu.VMEM),
            pl.BlockSpec(memory_space=pltpu.VMEM),
        ],
        out_specs=pl.BlockSpec(memory_space=pltpu.VMEM),
        scratch_shapes=[
            pltpu.VMEM((SLOTS, m_per, k), jnp.bfloat16),
            pltpu.VMEM((SLOTS, m_per, k), jnp.bfloat16),
            pltpu.VMEM((k, n_per), jnp.bfloat16),
            pltpu.SemaphoreType.DMA((SLOTS,)),
            pltpu.SemaphoreType.DMA((SLOTS,)),
            pltpu.SemaphoreType.DMA((SLOTS,)),
            pltpu.SemaphoreType.DMA((SLOTS,)),
        ],
        compiler_params=pltpu.CompilerParams(collective_id=0),
    )(x, w_mat)


# device time: 775449 ns/iter; 1.9444x vs baseline; 1.9041x over previous
import jax
import jax.numpy as jnp
from jax import lax
from jax.experimental import pallas as pl
from jax.experimental.pallas import tpu as pltpu

N_DEV = 32
R_STEPS = 16
L_STEPS = 15
SLOTS = 3
DRAIN_LAG = 2


def kernel(x, w_mat):
    m_x, k = x.shape
    _, n_per = w_mat.shape
    m_total = N_DEV * m_x

    def body(x_ref, w_ref, out_ref, comm_r, comm_l, x_bf,
             res_src_r, res_stage_r, res_src_l, res_stage_l,
             send_r, recv_r, send_l, recv_l,
             rsend_r, rrecv_r, rsend_l, rrecv_l):
        my = lax.axis_index("i")
        left = lax.rem(my - 1 + N_DEV, N_DEV)
        right = lax.rem(my + 1, N_DEV)

        barrier_sem = pltpu.get_barrier_semaphore()
        for nbr in (left, right):
            pl.semaphore_signal(
                barrier_sem, inc=1,
                device_id=(nbr,), device_id_type=pl.DeviceIdType.MESH,
            )
        pl.semaphore_wait(barrier_sem, 2)

        x_bf[...] = x_ref[...].astype(jnp.bfloat16)
        wb = w_ref[...].astype(jnp.bfloat16)
        comm_r[0] = wb
        comm_l[0] = wb

        def rdma_r(s):
            return pltpu.make_async_remote_copy(
                src_ref=comm_r.at[s % SLOTS],
                dst_ref=comm_r.at[(s + 1) % SLOTS],
                send_sem=send_r.at[s % SLOTS],
                recv_sem=recv_r.at[(s + 1) % SLOTS],
                device_id=(right,),
                device_id_type=pl.DeviceIdType.MESH,
            )

        def rdma_l(s):
            return pltpu.make_async_remote_copy(
                src_ref=comm_l.at[s % SLOTS],
                dst_ref=comm_l.at[(s + 1) % SLOTS],
                send_sem=send_l.at[s % SLOTS],
                recv_sem=recv_l.at[(s + 1) % SLOTS],
                device_id=(left,),
                device_id_type=pl.DeviceIdType.MESH,
            )

        def result_rdma_r(s, dest):
            return pltpu.make_async_remote_copy(
                src_ref=res_src_r.at[s],
                dst_ref=res_stage_r.at[s],
                send_sem=rsend_r.at[s],
                recv_sem=rrecv_r.at[s],
                device_id=(dest,),
                device_id_type=pl.DeviceIdType.MESH,
            )

        def result_rdma_l(s, dest):
            return pltpu.make_async_remote_copy(
                src_ref=res_src_l.at[s],
                dst_ref=res_stage_l.at[s],
                send_sem=rsend_l.at[s],
                recv_sem=rrecv_l.at[s],
                device_id=(dest,),
                device_id_type=pl.DeviceIdType.MESH,
            )

        def block_for(w_chunk):
            blk = jnp.dot(x_bf[...], w_chunk,
                          preferred_element_type=jnp.float32)
            return jnp.maximum(blk, 0.0)

        def drain_r(s):
            desc = result_rdma_r(s, left)
            desc.wait_send()
            desc.wait_recv()
            origin = lax.rem(my + s, N_DEV)
            out_ref[pl.ds(origin * m_x, m_x), :] = (
                res_stage_r[s].astype(jnp.float32))

        def drain_l(s):
            desc = result_rdma_l(s, right)
            desc.wait_send()
            desc.wait_recv()
            origin = lax.rem(my - s + N_DEV, N_DEV)
            out_ref[pl.ds(origin * m_x, m_x), :] = (
                res_stage_l[s].astype(jnp.float32))

        for s in range(R_STEPS):
            rr = rdma_r(s)
            rr.start()
            rl = None
            if s < L_STEPS:
                rl = rdma_l(s)
                rl.start()

            if s == 0:
                out_ref[pl.ds(my * m_x, m_x), :] = block_for(comm_r[0])
            else:
                o_r = lax.rem(my - s + N_DEV, N_DEV)
                res_src_r[s] = block_for(comm_r[s % SLOTS]).astype(jnp.bfloat16)
                result_rdma_r(s, o_r).start()
                o_l = lax.rem(my + s, N_DEV)
                res_src_l[s] = block_for(comm_l[s % SLOTS]).astype(jnp.bfloat16)
                result_rdma_l(s, o_l).start()

            if s >= DRAIN_LAG + 1:
                drain_r(s - DRAIN_LAG)
                drain_l(s - DRAIN_LAG)

            rr.wait()
            if rl is not None:
                rl.wait()

        o_r = lax.rem(my - R_STEPS + N_DEV, N_DEV)
        res_src_r[R_STEPS] = block_for(
            comm_r[R_STEPS % SLOTS]).astype(jnp.bfloat16)
        result_rdma_r(R_STEPS, o_r).start()

        for s in range(R_STEPS - DRAIN_LAG, R_STEPS + 1):
            drain_r(s)
        for s in range(R_STEPS - DRAIN_LAG, L_STEPS + 1):
            drain_l(s)

    return pl.pallas_call(
        body,
        out_shape=jax.ShapeDtypeStruct((m_total, n_per), jnp.float32),
        in_specs=[
            pl.BlockSpec(memory_space=pltpu.VMEM),
            pl.BlockSpec(memory_space=pltpu.VMEM),
        ],
        out_specs=pl.BlockSpec(memory_space=pltpu.VMEM),
        scratch_shapes=[
            pltpu.VMEM((SLOTS, k, n_per), jnp.bfloat16),
            pltpu.VMEM((SLOTS, k, n_per), jnp.bfloat16),
            pltpu.VMEM((m_x, k), jnp.bfloat16),
            pltpu.VMEM((R_STEPS + 1, m_x, n_per), jnp.bfloat16),
            pltpu.VMEM((R_STEPS + 1, m_x, n_per), jnp.bfloat16),
            pltpu.VMEM((L_STEPS + 1, m_x, n_per), jnp.bfloat16),
            pltpu.VMEM((L_STEPS + 1, m_x, n_per), jnp.bfloat16),
            pltpu.SemaphoreType.DMA((SLOTS,)),
            pltpu.SemaphoreType.DMA((SLOTS,)),
            pltpu.SemaphoreType.DMA((SLOTS,)),
            pltpu.SemaphoreType.DMA((SLOTS,)),
            pltpu.SemaphoreType.DMA((R_STEPS + 1,)),
            pltpu.SemaphoreType.DMA((R_STEPS + 1,)),
            pltpu.SemaphoreType.DMA((L_STEPS + 1,)),
            pltpu.SemaphoreType.DMA((L_STEPS + 1,)),
        ],
        compiler_params=pltpu.CompilerParams(collective_id=0),
    )(x, w_mat)
